# baseline (speedup 1.0000x reference)
import functools

import jax
import jax.numpy as jnp
from jax import lax
from jax.experimental import pallas as pl
from jax.experimental.pallas import tpu as pltpu
from jax.experimental.pallas import tpu_sc as plsc

_INFO = plsc.get_sparse_core_info()
_NC = _INFO.num_cores
_NS = _INFO.num_subcores
_NW = _NC * _NS
_L = _INFO.num_lanes


@functools.partial(jax.jit, static_argnames=("batch", "seq", "d", "n"))
def _sc_gather(x2, index, *, batch, seq, d, n):
    total = batch * n
    rows_per_w = total // _NW

    mesh = plsc.VectorSubcoreMesh(core_axis_name="c", subcore_axis_name="s")

    half = rows_per_w // 2

    @functools.partial(
        pl.kernel,
        mesh=mesh,
        out_type=jax.ShapeDtypeStruct((total, d), jnp.float32),
        scratch_types=[
            pltpu.VMEM((rows_per_w,), jnp.int32),
            pltpu.VMEM((rows_per_w,), jnp.int32),
            pltpu.VMEM((half, d), jnp.float32),
            pltpu.VMEM((half, d), jnp.float32),
            pltpu.SemaphoreType.DMA,
            pltpu.SemaphoreType.DMA,
            pltpu.SemaphoreType.DMA,
            pltpu.SemaphoreType.DMA,
        ],
    )
    def k(x_hbm, idx_hbm, out_hbm, idx_v, rid_v, rows0, rows1, g0s, g1s, s0s, s1s):
        wid = lax.axis_index("s") * _NC + lax.axis_index("c")
        base = wid * rows_per_w
        b = base // n
        pos = base - b * n
        pltpu.sync_copy(idx_hbm.at[pl.ds(pos, rows_per_w)], idx_v)
        rid_v[...] = idx_v[...] + b * seq
        g0 = pltpu.async_copy(x_hbm.at[rid_v.at[pl.ds(0, half)]], rows0, g0s)
        g1 = pltpu.async_copy(x_hbm.at[rid_v.at[pl.ds(half, half)]], rows1, g1s)
        g0.wait()
        s0 = pltpu.async_copy(rows0, out_hbm.at[pl.ds(base, half)], s0s)
        g1.wait()
        s1 = pltpu.async_copy(rows1, out_hbm.at[pl.ds(base + half, half)], s1s)
        s0.wait()
        s1.wait()

    return k(x2, index)


def kernel(x, index):
    batch, seq, d = x.shape
    n = index.shape[0]
    x2 = x.reshape(batch * seq, d)
    out = _sc_gather(x2, index, batch=batch, seq=seq, d=d, n=n)
    return out.reshape(batch, n, d)


# speedup vs baseline: 1.0059x; 1.0059x over previous
import functools

import jax
import jax.numpy as jnp
from jax import lax
from jax.experimental import pallas as pl
from jax.experimental.pallas import tpu as pltpu
from jax.experimental.pallas import tpu_sc as plsc

_INFO = plsc.get_sparse_core_info()
_NS = _INFO.num_subcores
_L = _INFO.num_lanes


@functools.partial(jax.jit, static_argnames=("batch", "seq", "d", "n"))
def _sc_gather(x2, index, *, batch, seq, d, n):
    total = batch * n
    rows_per_w = total // _NS
    half = rows_per_w // 2

    mesh = plsc.VectorSubcoreMesh(
        core_axis_name="c", subcore_axis_name="s", num_cores=1
    )

    @functools.partial(
        pl.kernel,
        mesh=mesh,
        out_type=jax.ShapeDtypeStruct((total, d), jnp.float32),
        scratch_types=[
            pltpu.VMEM((rows_per_w,), jnp.int32),
            pltpu.VMEM((rows_per_w,), jnp.int32),
            pltpu.VMEM((half, d), jnp.float32),
            pltpu.VMEM((half, d), jnp.float32),
            pltpu.SemaphoreType.DMA,
            pltpu.SemaphoreType.DMA,
            pltpu.SemaphoreType.DMA,
            pltpu.SemaphoreType.DMA,
        ],
    )
    def k(x_hbm, idx_hbm, out_hbm, idx_v, rid_v, rows0, rows1, g0s, g1s, s0s, s1s):
        wid = lax.axis_index("s")
        base = wid * rows_per_w
        b = base // n
        pos = base - b * n
        pltpu.sync_copy(idx_hbm.at[pl.ds(pos, rows_per_w)], idx_v)
        off = b * seq
        rid_v[pl.ds(0, _L)] = idx_v[pl.ds(0, _L)] + off
        rid_v[pl.ds(_L, _L)] = idx_v[pl.ds(_L, _L)] + off
        g0 = pltpu.async_copy(x_hbm.at[rid_v.at[pl.ds(0, half)]], rows0, g0s)
        g1 = pltpu.async_copy(x_hbm.at[rid_v.at[pl.ds(half, half)]], rows1, g1s)
        g0.wait()
        s0 = pltpu.async_copy(rows0, out_hbm.at[pl.ds(base, half)], s0s)
        g1.wait()
        s1 = pltpu.async_copy(rows1, out_hbm.at[pl.ds(base + half, half)], s1s)
        s0.wait()
        s1.wait()

    return k(x2, index)


def kernel(x, index):
    batch, seq, d = x.shape
    n = index.shape[0]
    x2 = x.reshape(batch * seq, d)
    out = _sc_gather(x2, index, batch=batch, seq=seq, d=d, n=n)
    return out.reshape(batch, n, d)


# speedup vs baseline: 1.0086x; 1.0027x over previous
import functools

import jax
import jax.numpy as jnp
from jax import lax
from jax.experimental import pallas as pl
from jax.experimental.pallas import tpu as pltpu
from jax.experimental.pallas import tpu_sc as plsc

_INFO = plsc.get_sparse_core_info()
_NS = _INFO.num_subcores
_L = _INFO.num_lanes

_NCHUNK = 4


@functools.partial(jax.jit, static_argnames=("batch", "seq", "d", "n"))
def _sc_gather(x2, index, *, batch, seq, d, n):
    total = batch * n
    rows_per_w = total // _NS
    chunk = rows_per_w // _NCHUNK

    mesh = plsc.VectorSubcoreMesh(
        core_axis_name="c", subcore_axis_name="s", num_cores=1
    )

    @functools.partial(
        pl.kernel,
        mesh=mesh,
        out_type=jax.ShapeDtypeStruct((total, d), jnp.float32),
        scratch_types=[
            pltpu.VMEM((rows_per_w,), jnp.int32),
            pltpu.VMEM((rows_per_w,), jnp.int32),
            *[pltpu.VMEM((chunk, d), jnp.float32) for _ in range(_NCHUNK)],
            *[pltpu.SemaphoreType.DMA for _ in range(2 * _NCHUNK)],
        ],
    )
    def k(x_hbm, idx_hbm, out_hbm, idx_v, rid_v, *bufs_and_sems):
        rows = bufs_and_sems[:_NCHUNK]
        gsems = bufs_and_sems[_NCHUNK:2 * _NCHUNK]
        ssems = bufs_and_sems[2 * _NCHUNK:]
        wid = lax.axis_index("s")
        base = wid * rows_per_w
        b = base // n
        pos = base - b * n
        pltpu.sync_copy(idx_hbm.at[pl.ds(pos, rows_per_w)], idx_v)
        off = b * seq
        for v in range(rows_per_w // _L):
            rid_v[pl.ds(v * _L, _L)] = idx_v[pl.ds(v * _L, _L)] + off
        gathers = [
            pltpu.async_copy(
                x_hbm.at[rid_v.at[pl.ds(c * chunk, chunk)]], rows[c], gsems[c]
            )
            for c in range(_NCHUNK)
        ]
        scatters = []
        for c in range(_NCHUNK):
            gathers[c].wait()
            scatters.append(
                pltpu.async_copy(
                    rows[c], out_hbm.at[pl.ds(base + c * chunk, chunk)], ssems[c]
                )
            )
        for s in scatters:
            s.wait()

    return k(x2, index)


def kernel(x, index):
    batch, seq, d = x.shape
    n = index.shape[0]
    x2 = x.reshape(batch * seq, d)
    out = _sc_gather(x2, index, batch=batch, seq=seq, d=d, n=n)
    return out.reshape(batch, n, d)
